# baseline (speedup 1.0000x reference)
import functools

import jax
import jax.numpy as jnp
from jax import lax
from jax.experimental import pallas as pl
from jax.experimental.pallas import tpu as pltpu
from jax.experimental.pallas import tpu_sc as plsc

_VOCAB = 1000000
_DIM = 64
_SEQ = 200
_NC = 2
_NS = 16
_NW = _NC * _NS
_L = 16
_TR = _SEQ // 8
_VT = _VOCAB // 128
_VREM = _VOCAB - _VT * 128
_PW = _VT % _NW
_STRIPE = 128 * _DIM


def _mesh():
    return plsc.VectorSubcoreMesh(
        core_axis_name="c", subcore_axis_name="s",
        num_cores=_NC, num_subcores=_NS)


def _wid():
    return lax.axis_index("s") * _NC + lax.axis_index("c")


def _detile_body(textT_hbm, idx_hbm, stage_v, sem):
    wid = _wid()
    cps = [
        pltpu.async_copy(
            textT_hbm.at[pl.ds(i * 8, 8), pl.ds(wid * 128, 128)],
            stage_v.at[i], sem)
        for i in range(_TR)
    ]
    for cp in cps:
        cp.wait()
    pltpu.sync_copy(stage_v, idx_hbm.at[pl.ds(wid * _TR, _TR)])


def _transpose_body(tT_hbm, comp_hbm, in_a, in_b, out_a, out_b, inp_v,
                    outp_v, sem_i, sem_o):
    wid = _wid()
    n_full = (_VT - 1 - wid) // _NW + 1
    lane64 = lax.iota(jnp.int32, _L) * _DIM
    scale = jnp.float32(1.0 / _SEQ)

    def fire_in(vt, buf):
        pltpu.async_copy(tT_hbm.at[:, pl.ds(vt * 128, 128)], buf, sem_i)

    def drain_in(buf):
        pltpu.make_async_copy(tT_hbm.at[:, pl.ds(0, 128)], buf,
                              sem_i).wait()

    def drain_out():
        pltpu.make_async_copy(comp_hbm.at[pl.ds(0, _STRIPE)],
                              out_a, sem_o).wait()

    def step(vt, src_v, dst_v):
        def tb(d, _):
            for c in range(128 // _L):
                x = src_v[d, pl.ds(c * _L, _L)] * scale
                plsc.store_scatter(dst_v,
                                   [lane64 + (c * _L * _DIM + d)], x)
            return 0

        lax.fori_loop(0, _DIM, tb, 0)
        pltpu.async_copy(dst_v,
                         comp_hbm.at[pl.ds(vt * _STRIPE, _STRIPE)], sem_o)

    fire_in(wid, in_a)

    def body(t, _):
        vt = wid + t * _NW
        even = lax.rem(t, 2) == 0

        @pl.when(t + 1 < n_full)
        def _():
            @pl.when(even)
            def _():
                fire_in(vt + _NW, in_b)

            @pl.when(jnp.logical_not(even))
            def _():
                fire_in(vt + _NW, in_a)

        @pl.when(t >= 2)
        def _():
            drain_out()

        @pl.when(even)
        def _():
            drain_in(in_a)
            step(vt, in_a, out_a)

        @pl.when(jnp.logical_not(even))
        def _():
            drain_in(in_b)
            step(vt, in_b, out_b)

        return 0

    lax.fori_loop(0, n_full, body, 0)
    drain_out()

    @pl.when(n_full >= 2)
    def _():
        drain_out()

    @pl.when(wid == _PW)
    def _():
        pltpu.sync_copy(tT_hbm.at[:, pl.ds(_VT * 128, _VREM)], inp_v)

        def tbp(d, _):
            for c in range(_VREM // _L):
                x = inp_v[d, pl.ds(c * _L, _L)] * scale
                plsc.store_scatter(outp_v,
                                   [lane64 + (c * _L * _DIM + d)], x)
            return 0

        lax.fori_loop(0, _DIM, tbp, 0)
        pltpu.sync_copy(
            outp_v, comp_hbm.at[pl.ds(_VT * _STRIPE, _VREM * _DIM)])


def _emb_mean_body(bpw, idx_hbm, table_hbm, out_hbm, idx_v, acc_v, sem):
    wid = _wid()
    b0 = wid * bpw

    pltpu.sync_copy(idx_hbm.at[pl.ds(wid * _TR, _TR)], idx_v)

    def zbody(r, _):
        for k in range(_DIM // _L):
            acc_v[r, pl.ds(k * _L, _L)] = jnp.zeros((_L,), jnp.float32)
        return 0

    lax.fori_loop(0, bpw, zbody, 0)

    def fire(i):
        for r in range(8):
            pltpu.async_copy(table_hbm.at[idx_v.at[i, r]], acc_v, sem,
                             add=True)

    def drain():
        for _ in range(8):
            pltpu.make_async_copy(table_hbm.at[pl.ds(0, bpw)], acc_v,
                                  sem).wait()

    fire(0)

    def body(i, _):
        fire(i)
        drain()
        return 0

    lax.fori_loop(1, _TR, body, 0)
    drain()
    pltpu.sync_copy(acc_v, out_hbm.at[pl.ds(b0, bpw)])


@functools.partial(jax.jit, static_argnames=("batch",))
def _emb_mean(textT, tableT, batch):
    bpw = batch // _NW
    idx = pl.kernel(
        _detile_body,
        out_type=jax.ShapeDtypeStruct((_NW * _TR, 8, 128), jnp.int32),
        mesh=_mesh(),
        compiler_params=pltpu.CompilerParams(use_tc_tiling_on_sc=True),
        scratch_types=[
            pltpu.VMEM((_TR, 8, 128), jnp.int32),
            pltpu.SemaphoreType.DMA,
        ],
    )(textT)
    comp = pl.kernel(
        _transpose_body,
        out_type=jax.ShapeDtypeStruct((_VOCAB * _DIM,), jnp.float32),
        mesh=_mesh(),
        compiler_params=pltpu.CompilerParams(
            use_tc_tiling_on_sc=True, needs_layout_passes=False),
        scratch_types=[
            pltpu.VMEM((_DIM, 128), jnp.float32),
            pltpu.VMEM((_DIM, 128), jnp.float32),
            pltpu.VMEM((_STRIPE,), jnp.float32),
            pltpu.VMEM((_STRIPE,), jnp.float32),
            pltpu.VMEM((_DIM, _VREM), jnp.float32),
            pltpu.VMEM((_VREM * _DIM,), jnp.float32),
            pltpu.SemaphoreType.DMA,
            pltpu.SemaphoreType.DMA,
        ],
    )(tableT)
    rm = jnp.reshape(comp, (_VOCAB, _DIM))
    return pl.kernel(
        functools.partial(_emb_mean_body, bpw),
        out_type=jax.ShapeDtypeStruct((batch, _DIM), jnp.float32),
        mesh=_mesh(),
        compiler_params=pltpu.CompilerParams(use_tc_tiling_on_sc=False),
        scratch_types=[
            pltpu.VMEM((_TR, 8, 128), jnp.int32),
            pltpu.VMEM((bpw, _DIM), jnp.float32),
            pltpu.SemaphoreType.DMA,
        ],
    )(idx, rm)


def kernel(text, table):
    batch = text.shape[0]
    textT = jnp.swapaxes(text.astype(jnp.int32), 0, 1)
    tableT = jnp.swapaxes(table, 0, 1)
    return _emb_mean(textT, tableT, batch)


# speedup vs baseline: 2.0478x; 2.0478x over previous
import functools

import jax
import jax.numpy as jnp
from jax import lax
from jax.experimental import pallas as pl
from jax.experimental.pallas import tpu as pltpu
from jax.experimental.pallas import tpu_sc as plsc

_VOCAB = 1000000
_DIM = 64
_SEQ = 200
_NC = 2
_NS = 16
_NW = _NC * _NS
_L = 16
_TR = _SEQ // 8
_VT = _VOCAB // 128
_VREM = _VOCAB - _VT * 128
_PW = _VT % _NW
_STRIPE = 128 * _DIM


def _mesh():
    return plsc.VectorSubcoreMesh(
        core_axis_name="c", subcore_axis_name="s",
        num_cores=_NC, num_subcores=_NS)


def _wid():
    return lax.axis_index("s") * _NC + lax.axis_index("c")


def _detile_body(textT_hbm, idx_hbm, stage_v, sem):
    wid = _wid()
    cps = [
        pltpu.async_copy(
            textT_hbm.at[pl.ds(i * 8, 8), pl.ds(wid * 128, 128)],
            stage_v.at[i], sem)
        for i in range(_TR)
    ]
    for cp in cps:
        cp.wait()
    pltpu.sync_copy(stage_v, idx_hbm.at[pl.ds(wid * _TR, _TR)])


def _transpose_body(tT_hbm, comp_hbm, in_a, in_b, out_a, out_b, inp_v,
                    outp_v, sem_i, sem_o):
    wid = _wid()
    n_full = (_VT - 1 - wid) // _NW + 1
    lane = lax.iota(jnp.int32, _L)
    scale = jnp.float32(1.0 / _SEQ)

    def fire_in(vt, buf):
        pltpu.async_copy(tT_hbm.at[:, pl.ds(vt * 128, 128)], buf, sem_i)

    def drain_in(buf):
        pltpu.make_async_copy(tT_hbm.at[:, pl.ds(0, 128)], buf,
                              sem_i).wait()

    def drain_out():
        pltpu.make_async_copy(comp_hbm.at[pl.ds(0, _STRIPE)],
                              out_a, sem_o).wait()

    def step(vt, src_v, dst_v):
        def sb(s, _):
            m = lax.bitwise_and(lane + s, _L - 1)
            dstb = m * _DIM + lane
            for d0 in range(_DIM // _L):
                for v0 in range(128 // _L):
                    g = plsc.load_gather(
                        src_v, [lane + d0 * _L, m + v0 * _L]) * scale
                    plsc.store_scatter(
                        dst_v, [dstb + (v0 * _L * _DIM + d0 * _L)], g)
            return 0

        lax.fori_loop(0, _L, sb, 0)
        pltpu.async_copy(dst_v,
                         comp_hbm.at[pl.ds(vt * _STRIPE, _STRIPE)], sem_o)

    fire_in(wid, in_a)

    def body(t, _):
        vt = wid + t * _NW
        even = lax.rem(t, 2) == 0

        @pl.when(t + 1 < n_full)
        def _():
            @pl.when(even)
            def _():
                fire_in(vt + _NW, in_b)

            @pl.when(jnp.logical_not(even))
            def _():
                fire_in(vt + _NW, in_a)

        @pl.when(t >= 2)
        def _():
            drain_out()

        @pl.when(even)
        def _():
            drain_in(in_a)
            step(vt, in_a, out_a)

        @pl.when(jnp.logical_not(even))
        def _():
            drain_in(in_b)
            step(vt, in_b, out_b)

        return 0

    lax.fori_loop(0, n_full, body, 0)
    drain_out()

    @pl.when(n_full >= 2)
    def _():
        drain_out()

    @pl.when(wid == _PW)
    def _():
        pltpu.sync_copy(tT_hbm.at[:, pl.ds(_VT * 128, _VREM)], inp_v)

        def sbp(s, _):
            m = lax.bitwise_and(lane + s, _L - 1)
            dstb = m * _DIM + lane
            for d0 in range(_DIM // _L):
                for v0 in range(_VREM // _L):
                    g = plsc.load_gather(
                        inp_v, [lane + d0 * _L, m + v0 * _L]) * scale
                    plsc.store_scatter(
                        outp_v, [dstb + (v0 * _L * _DIM + d0 * _L)], g)
            return 0

        lax.fori_loop(0, _L, sbp, 0)
        pltpu.sync_copy(
            outp_v, comp_hbm.at[pl.ds(_VT * _STRIPE, _VREM * _DIM)])


def _emb_mean_body(bpw, idx_hbm, table_hbm, out_hbm, idx_v, acc_v, sem):
    wid = _wid()
    b0 = wid * bpw

    pltpu.sync_copy(idx_hbm.at[pl.ds(wid * _TR, _TR)], idx_v)

    def zbody(r, _):
        for k in range(_DIM // _L):
            acc_v[r, pl.ds(k * _L, _L)] = jnp.zeros((_L,), jnp.float32)
        return 0

    lax.fori_loop(0, bpw, zbody, 0)

    def fire(i):
        for r in range(8):
            pltpu.async_copy(table_hbm.at[idx_v.at[i, r]], acc_v, sem,
                             add=True)

    def drain():
        for _ in range(8):
            pltpu.make_async_copy(table_hbm.at[pl.ds(0, bpw)], acc_v,
                                  sem).wait()

    fire(0)

    def body(i, _):
        fire(i)
        drain()
        return 0

    lax.fori_loop(1, _TR, body, 0)
    drain()
    pltpu.sync_copy(acc_v, out_hbm.at[pl.ds(b0, bpw)])


@functools.partial(jax.jit, static_argnames=("batch",))
def _emb_mean(textT, tableT, batch):
    bpw = batch // _NW
    idx = pl.kernel(
        _detile_body,
        out_type=jax.ShapeDtypeStruct((_NW * _TR, 8, 128), jnp.int32),
        mesh=_mesh(),
        compiler_params=pltpu.CompilerParams(use_tc_tiling_on_sc=True),
        scratch_types=[
            pltpu.VMEM((_TR, 8, 128), jnp.int32),
            pltpu.SemaphoreType.DMA,
        ],
    )(textT)
    comp = pl.kernel(
        _transpose_body,
        out_type=jax.ShapeDtypeStruct((_VOCAB * _DIM,), jnp.float32),
        mesh=_mesh(),
        compiler_params=pltpu.CompilerParams(
            use_tc_tiling_on_sc=True, needs_layout_passes=False),
        scratch_types=[
            pltpu.VMEM((_DIM, 128), jnp.float32),
            pltpu.VMEM((_DIM, 128), jnp.float32),
            pltpu.VMEM((_STRIPE,), jnp.float32),
            pltpu.VMEM((_STRIPE,), jnp.float32),
            pltpu.VMEM((_DIM, _VREM), jnp.float32),
            pltpu.VMEM((_VREM * _DIM,), jnp.float32),
            pltpu.SemaphoreType.DMA,
            pltpu.SemaphoreType.DMA,
        ],
    )(tableT)
    rm = jnp.reshape(comp, (_VOCAB, _DIM))
    return pl.kernel(
        functools.partial(_emb_mean_body, bpw),
        out_type=jax.ShapeDtypeStruct((batch, _DIM), jnp.float32),
        mesh=_mesh(),
        compiler_params=pltpu.CompilerParams(use_tc_tiling_on_sc=False),
        scratch_types=[
            pltpu.VMEM((_TR, 8, 128), jnp.int32),
            pltpu.VMEM((bpw, _DIM), jnp.float32),
            pltpu.SemaphoreType.DMA,
        ],
    )(idx, rm)


def kernel(text, table):
    batch = text.shape[0]
    textT = jnp.swapaxes(text.astype(jnp.int32), 0, 1)
    tableT = jnp.swapaxes(table, 0, 1)
    return _emb_mean(textT, tableT, batch)


# speedup vs baseline: 2.0649x; 1.0084x over previous
import functools

import jax
import jax.numpy as jnp
from jax import lax
from jax.experimental import pallas as pl
from jax.experimental.pallas import tpu as pltpu
from jax.experimental.pallas import tpu_sc as plsc

_VOCAB = 1000000
_DIM = 64
_SEQ = 200
_NC = 2
_NS = 16
_NW = _NC * _NS
_L = 16
_TR = _SEQ // 8
_VT = _VOCAB // 128
_VREM = _VOCAB - _VT * 128
_PW = _VT % _NW
_STRIPE = 128 * _DIM


def _mesh():
    return plsc.VectorSubcoreMesh(
        core_axis_name="c", subcore_axis_name="s",
        num_cores=_NC, num_subcores=_NS)


def _wid():
    return lax.axis_index("s") * _NC + lax.axis_index("c")


def _detile_body(textT_hbm, idx_hbm, stage_v, sem):
    wid = _wid()
    cps = [
        pltpu.async_copy(
            textT_hbm.at[pl.ds(i * 8, 8), pl.ds(wid * 128, 128)],
            stage_v.at[i], sem)
        for i in range(_TR)
    ]
    for cp in cps:
        cp.wait()
    pltpu.sync_copy(stage_v, idx_hbm.at[pl.ds(wid * _TR, _TR)])


def _transpose_body(tT_hbm, comp_hbm, in_a, in_b, out_a, out_b, inp_v,
                    outp_v, sem_i, sem_o):
    wid = _wid()
    n_full = (_VT - 1 - wid) // _NW + 1
    lane = lax.iota(jnp.int32, _L)
    scale = jnp.float32(1.0 / _SEQ)

    def fire_in(vt, buf):
        pltpu.async_copy(tT_hbm.at[:, pl.ds(vt * 128, 128)], buf, sem_i)

    def drain_in(buf):
        pltpu.make_async_copy(tT_hbm.at[:, pl.ds(0, 128)], buf,
                              sem_i).wait()

    def drain_out():
        pltpu.make_async_copy(comp_hbm.at[pl.ds(0, _STRIPE)],
                              out_a, sem_o).wait()

    srcd = [lane + d0 * _L for d0 in range(_DIM // _L)]

    def step(vt, src_v, dst_v):
        def sb(s2, _):
            for u in range(2):
                s = s2 * 2 + u
                m = lax.bitwise_and(lane + s, _L - 1)
                dstb = m * _DIM + lane
                srcv = [m + v0 * _L for v0 in range(128 // _L)]
                for d0 in range(_DIM // _L):
                    for v0 in range(128 // _L):
                        g = plsc.load_gather(
                            src_v, [srcd[d0], srcv[v0]]) * scale
                        plsc.store_scatter(
                            dst_v, [dstb + (v0 * _L * _DIM + d0 * _L)], g)
            return 0

        lax.fori_loop(0, _L // 2, sb, 0)
        pltpu.async_copy(dst_v,
                         comp_hbm.at[pl.ds(vt * _STRIPE, _STRIPE)], sem_o)

    fire_in(wid, in_a)

    def body(t, _):
        vt = wid + t * _NW
        even = lax.rem(t, 2) == 0

        @pl.when(t + 1 < n_full)
        def _():
            @pl.when(even)
            def _():
                fire_in(vt + _NW, in_b)

            @pl.when(jnp.logical_not(even))
            def _():
                fire_in(vt + _NW, in_a)

        @pl.when(t >= 2)
        def _():
            drain_out()

        @pl.when(even)
        def _():
            drain_in(in_a)
            step(vt, in_a, out_a)

        @pl.when(jnp.logical_not(even))
        def _():
            drain_in(in_b)
            step(vt, in_b, out_b)

        return 0

    lax.fori_loop(0, n_full, body, 0)
    drain_out()

    @pl.when(n_full >= 2)
    def _():
        drain_out()

    @pl.when(wid == _PW)
    def _():
        pltpu.sync_copy(tT_hbm.at[:, pl.ds(_VT * 128, _VREM)], inp_v)

        def sbp(s, _):
            m = lax.bitwise_and(lane + s, _L - 1)
            dstb = m * _DIM + lane
            for d0 in range(_DIM // _L):
                for v0 in range(_VREM // _L):
                    g = plsc.load_gather(
                        inp_v, [lane + d0 * _L, m + v0 * _L]) * scale
                    plsc.store_scatter(
                        outp_v, [dstb + (v0 * _L * _DIM + d0 * _L)], g)
            return 0

        lax.fori_loop(0, _L, sbp, 0)
        pltpu.sync_copy(
            outp_v, comp_hbm.at[pl.ds(_VT * _STRIPE, _VREM * _DIM)])


def _emb_mean_body(bpw, idx_hbm, table_hbm, out_hbm, idx_v, acc_v, sem):
    wid = _wid()
    b0 = wid * bpw

    pltpu.sync_copy(idx_hbm.at[pl.ds(wid * _TR, _TR)], idx_v)

    def zbody(r, _):
        for k in range(_DIM // _L):
            acc_v[r, pl.ds(k * _L, _L)] = jnp.zeros((_L,), jnp.float32)
        return 0

    lax.fori_loop(0, bpw, zbody, 0)

    def fire(i):
        for r in range(8):
            pltpu.async_copy(table_hbm.at[idx_v.at[i, r]], acc_v, sem,
                             add=True)

    def drain():
        for _ in range(8):
            pltpu.make_async_copy(table_hbm.at[pl.ds(0, bpw)], acc_v,
                                  sem).wait()

    fire(0)

    def body(i, _):
        fire(i)
        drain()
        return 0

    lax.fori_loop(1, _TR, body, 0)
    drain()
    pltpu.sync_copy(acc_v, out_hbm.at[pl.ds(b0, bpw)])


@functools.partial(jax.jit, static_argnames=("batch",))
def _emb_mean(textT, tableT, batch):
    bpw = batch // _NW
    idx = pl.kernel(
        _detile_body,
        out_type=jax.ShapeDtypeStruct((_NW * _TR, 8, 128), jnp.int32),
        mesh=_mesh(),
        compiler_params=pltpu.CompilerParams(use_tc_tiling_on_sc=True),
        scratch_types=[
            pltpu.VMEM((_TR, 8, 128), jnp.int32),
            pltpu.SemaphoreType.DMA,
        ],
    )(textT)
    comp = pl.kernel(
        _transpose_body,
        out_type=jax.ShapeDtypeStruct((_VOCAB * _DIM,), jnp.float32),
        mesh=_mesh(),
        compiler_params=pltpu.CompilerParams(
            use_tc_tiling_on_sc=True, needs_layout_passes=False),
        scratch_types=[
            pltpu.VMEM((_DIM, 128), jnp.float32),
            pltpu.VMEM((_DIM, 128), jnp.float32),
            pltpu.VMEM((_STRIPE,), jnp.float32),
            pltpu.VMEM((_STRIPE,), jnp.float32),
            pltpu.VMEM((_DIM, _VREM), jnp.float32),
            pltpu.VMEM((_VREM * _DIM,), jnp.float32),
            pltpu.SemaphoreType.DMA,
            pltpu.SemaphoreType.DMA,
        ],
    )(tableT)
    rm = jnp.reshape(comp, (_VOCAB, _DIM))
    return pl.kernel(
        functools.partial(_emb_mean_body, bpw),
        out_type=jax.ShapeDtypeStruct((batch, _DIM), jnp.float32),
        mesh=_mesh(),
        compiler_params=pltpu.CompilerParams(use_tc_tiling_on_sc=False),
        scratch_types=[
            pltpu.VMEM((_TR, 8, 128), jnp.int32),
            pltpu.VMEM((bpw, _DIM), jnp.float32),
            pltpu.SemaphoreType.DMA,
        ],
    )(idx, rm)


def kernel(text, table):
    batch = text.shape[0]
    textT = jnp.swapaxes(text.astype(jnp.int32), 0, 1)
    tableT = jnp.swapaxes(table, 0, 1)
    return _emb_mean(textT, tableT, batch)
